# baseline (speedup 1.0000x reference)
import jax
import jax.numpy as jnp
from jax import lax
from jax.experimental import pallas as pl
from jax.experimental.pallas import tpu as pltpu
from jax.experimental.pallas import tpu_sc as plsc

VOCAB = 100000
SEQLEN = 2048
EMBED = 768
BATCH = 4
TOKENS = BATCH * SEQLEN

NC = 2
NS = 16
NW = NC * NS
SPW = SEQLEN // NW
CHUNK = 8
NQ = SPW // CHUNK
QROWS = BATCH * CHUNK
LANES = 16
VECS = EMBED // LANES
NB = 2


def _emb_body(ids_hbm, oidx_hbm, tok_hbm, pos_hbm, out_hbm,
              idx_v, oidx_v, pos_v, gat_v, isem, psem, gsem, osem):
    wid = lax.axis_index("s") * NC + lax.axis_index("c")
    sbase = wid * SPW

    pltpu.async_copy(ids_hbm.at[pl.ds(wid * NQ * QROWS, NQ * QROWS)],
                     idx_v, isem)
    pltpu.async_copy(oidx_hbm.at[wid], oidx_v, isem)
    pltpu.make_async_copy(ids_hbm.at[pl.ds(wid * NQ * QROWS, NQ * QROWS)],
                          idx_v, isem).wait()
    pltpu.make_async_copy(oidx_hbm.at[wid], oidx_v, isem).wait()

    def gather_desc(qd, s):
        return pltpu.make_async_copy(
            tok_hbm.at[idx_v.at[pl.ds(qd * QROWS, QROWS)]],
            gat_v.at[s], gsem.at[s])

    def pos_desc(qd):
        return pltpu.make_async_copy(
            pos_hbm.at[pl.ds(sbase + qd * CHUNK, CHUNK)],
            pos_v.at[pl.ds(qd * CHUNK, CHUNK)], psem)

    def out_desc(qd, s):
        return pltpu.make_async_copy(
            gat_v.at[s], out_hbm.at[oidx_v.at[qd]], osem.at[s])

    gather_desc(0, 0).start()
    pos_desc(0).start()
    for qd in range(1, NQ):
        pos_desc(qd).start()

    @pl.loop(0, NQ, step=NB)
    def quad_group(g):
        for i in range(NB):
            qd = g + i
            s = i
            sn = (i + 1) % NB

            @pl.when(qd + 1 < NQ)
            def _():
                @pl.when(qd >= 1)
                def _():
                    out_desc(qd - 1, sn).wait()
                gather_desc(qd + 1, sn).start()

            gather_desc(qd, s).wait()
            pos_desc(qd).wait()

            @plsc.parallel_loop(0, CHUNK, unroll=2)
            def add_row(k):
                pr = qd * CHUNK + k
                for j in range(VECS):
                    sl = pl.ds(j * LANES, LANES)
                    pv = pos_v[pr, sl]
                    for b in range(BATCH):
                        plsc.addupdate(gat_v.at[s, b * CHUNK + k, sl], pv)

            out_desc(qd, s).start()

    out_desc(NQ - 2, 0).wait()
    out_desc(NQ - 1, 1).wait()


@jax.jit
def _emb_call(ids_perm, oidx, token_embeddings, position_embeddings):
    mesh = plsc.VectorSubcoreMesh(core_axis_name="c", subcore_axis_name="s")
    return pl.kernel(
        _emb_body,
        out_type=jax.ShapeDtypeStruct((TOKENS, EMBED), jnp.float32),
        mesh=mesh,
        scratch_types=[
            pltpu.VMEM((NQ * QROWS,), jnp.int32),
            pltpu.VMEM((NQ, QROWS), jnp.int32),
            pltpu.VMEM((SPW, EMBED), jnp.float32),
            pltpu.VMEM((NB, QROWS, EMBED), jnp.float32),
            pltpu.SemaphoreType.DMA,
            pltpu.SemaphoreType.DMA,
            pltpu.SemaphoreType.DMA((NB,)),
            pltpu.SemaphoreType.DMA((NB,)),
        ],
    )(ids_perm, oidx, token_embeddings, position_embeddings)


def kernel(input_ids, token_embeddings, position_embeddings):
    ids2 = input_ids.astype(jnp.int32).reshape(BATCH, NW, NQ, CHUNK)
    ids_perm = ids2.transpose(1, 2, 0, 3).reshape(-1)
    w_ = jnp.arange(NW, dtype=jnp.int32).reshape(NW, 1, 1, 1)
    q_ = jnp.arange(NQ, dtype=jnp.int32).reshape(1, NQ, 1, 1)
    b_ = jnp.arange(BATCH, dtype=jnp.int32).reshape(1, 1, BATCH, 1)
    k_ = jnp.arange(CHUNK, dtype=jnp.int32).reshape(1, 1, 1, CHUNK)
    oidx = (b_ * SEQLEN + w_ * SPW + q_ * CHUNK + k_).reshape(NW, NQ, QROWS)
    out = _emb_call(ids_perm, oidx, token_embeddings, position_embeddings)
    return out.reshape(BATCH, SEQLEN, EMBED)


# speedup vs baseline: 1.0213x; 1.0213x over previous
import jax
import jax.numpy as jnp
from jax import lax
from jax.experimental import pallas as pl
from jax.experimental.pallas import tpu as pltpu
from jax.experimental.pallas import tpu_sc as plsc

VOCAB = 100000
SEQLEN = 2048
EMBED = 768
BATCH = 4
TOKENS = BATCH * SEQLEN

NC = 2
NS = 16
NW = NC * NS
SPW = SEQLEN // NW
CHUNK = 8
NQ = SPW // CHUNK
QROWS = BATCH * CHUNK
LANES = 16
VECS = EMBED // LANES
NB = 2


def _emb_body(ids_hbm, oidx_hbm, tok_hbm, pos_hbm, out_hbm,
              idx_v, oidx_v, pos_v, gat_v, isem, psem, gsem, osem):
    wid = lax.axis_index("s") * NC + lax.axis_index("c")
    sbase = wid * SPW

    pltpu.async_copy(ids_hbm.at[pl.ds(wid * NQ * QROWS, NQ * QROWS)],
                     idx_v, isem)
    pltpu.async_copy(oidx_hbm.at[wid], oidx_v, isem)
    ppend = pltpu.async_copy(pos_hbm.at[pl.ds(sbase, SPW)], pos_v, psem)
    pltpu.make_async_copy(ids_hbm.at[pl.ds(wid * NQ * QROWS, NQ * QROWS)],
                          idx_v, isem).wait()
    pltpu.make_async_copy(oidx_hbm.at[wid], oidx_v, isem).wait()

    def gather_desc(qd, s):
        return pltpu.make_async_copy(
            tok_hbm.at[idx_v.at[pl.ds(qd * QROWS, QROWS)]],
            gat_v.at[s], gsem.at[s])

    def out_desc(qd, s):
        return pltpu.make_async_copy(
            gat_v.at[s], out_hbm.at[oidx_v.at[qd]], osem.at[s])

    gather_desc(0, 0).start()
    ppend.wait()

    @pl.loop(0, NQ, step=NB)
    def quad_group(g):
        for i in range(NB):
            qd = g + i
            s = i
            sn = (i + 1) % NB

            @pl.when(qd + 1 < NQ)
            def _():
                @pl.when(qd >= 1)
                def _():
                    out_desc(qd - 1, sn).wait()
                gather_desc(qd + 1, sn).start()

            gather_desc(qd, s).wait()

            @plsc.parallel_loop(0, CHUNK, unroll=2)
            def add_row(k):
                pr = qd * CHUNK + k
                for j in range(VECS):
                    sl = pl.ds(j * LANES, LANES)
                    pv = pos_v[pr, sl]
                    for b in range(BATCH):
                        plsc.addupdate(gat_v.at[s, b * CHUNK + k, sl], pv)

            out_desc(qd, s).start()

    out_desc(NQ - 2, 0).wait()
    out_desc(NQ - 1, 1).wait()


@jax.jit
def _emb_call(ids_perm, oidx, token_embeddings, position_embeddings):
    mesh = plsc.VectorSubcoreMesh(core_axis_name="c", subcore_axis_name="s")
    return pl.kernel(
        _emb_body,
        out_type=jax.ShapeDtypeStruct((TOKENS, EMBED), jnp.float32),
        mesh=mesh,
        scratch_types=[
            pltpu.VMEM((NQ * QROWS,), jnp.int32),
            pltpu.VMEM((NQ, QROWS), jnp.int32),
            pltpu.VMEM((SPW, EMBED), jnp.float32),
            pltpu.VMEM((NB, QROWS, EMBED), jnp.float32),
            pltpu.SemaphoreType.DMA,
            pltpu.SemaphoreType.DMA,
            pltpu.SemaphoreType.DMA((NB,)),
            pltpu.SemaphoreType.DMA((NB,)),
        ],
    )(ids_perm, oidx, token_embeddings, position_embeddings)


def kernel(input_ids, token_embeddings, position_embeddings):
    ids2 = input_ids.astype(jnp.int32).reshape(BATCH, NW, NQ, CHUNK)
    ids_perm = ids2.transpose(1, 2, 0, 3).reshape(-1)
    w_ = jnp.arange(NW, dtype=jnp.int32).reshape(NW, 1, 1, 1)
    q_ = jnp.arange(NQ, dtype=jnp.int32).reshape(1, NQ, 1, 1)
    b_ = jnp.arange(BATCH, dtype=jnp.int32).reshape(1, 1, BATCH, 1)
    k_ = jnp.arange(CHUNK, dtype=jnp.int32).reshape(1, 1, 1, CHUNK)
    oidx = (b_ * SEQLEN + w_ * SPW + q_ * CHUNK + k_).reshape(NW, NQ, QROWS)
    out = _emb_call(ids_perm, oidx, token_embeddings, position_embeddings)
    return out.reshape(BATCH, SEQLEN, EMBED)
